# baseline (speedup 1.0000x reference)
import functools

import jax
import jax.numpy as jnp
from jax import lax
from jax.experimental import pallas as pl

_CODEBOOK = 1024
_DIM = 256
_BETA = 0.25

_BLK = 512
_NTOK = 8192
_NBLK = _NTOK // _BLK


def _vq_body(zf_ref, wt_ref, w_ref, zsq_ref, wsq_ref,
             zq_ref, idx_ref, loss_ref):
    i = pl.program_id(0)
    zblk = zf_ref[...]
    mm = jnp.dot(zblk, wt_ref[...],
                 preferred_element_type=jnp.float32)
    d = (zsq_ref[...] + wsq_ref[...]) - 2.0 * mm
    dmin = jnp.min(d, axis=1)
    iota = lax.broadcasted_iota(jnp.int32, (_BLK, _CODEBOOK), 1)
    idx = jnp.min(jnp.where(d == dmin[:, None], iota, _CODEBOOK), axis=1)
    idx_ref[...] = idx.reshape(1, 1, _BLK)

    @pl.when(i == 0)
    def _():
        loss_ref[...] = jnp.zeros_like(loss_ref)

    loss_ref[...] += jnp.sum(dmin).reshape(1, 1)

    onehot = (lax.broadcasted_iota(jnp.int32, (_BLK, _CODEBOOK), 1)
              == idx[:, None]).astype(jnp.float32)
    zq_ref[...] = jnp.dot(onehot, w_ref[...],
                          preferred_element_type=jnp.float32,
                          precision=lax.Precision.HIGHEST)


@jax.jit
def kernel(z, W):
    zp = jnp.transpose(z, (0, 2, 3, 1))
    zf = zp.reshape(-1, _DIM)
    zsq = jnp.sum(zf ** 2, axis=1, keepdims=True)
    wsq = jnp.sum(W ** 2, axis=1)
    wt = W.T

    zq_flat, idx3, loss_sum = pl.pallas_call(
        _vq_body,
        grid=(_NBLK,),
        in_specs=[
            pl.BlockSpec((_BLK, _DIM), lambda i: (i, 0)),
            pl.BlockSpec((_DIM, _CODEBOOK), lambda i: (0, 0)),
            pl.BlockSpec((_CODEBOOK, _DIM), lambda i: (0, 0)),
            pl.BlockSpec((_BLK, 1), lambda i: (i, 0)),
            pl.BlockSpec((1, _CODEBOOK), lambda i: (0, 0)),
        ],
        out_specs=[
            pl.BlockSpec((_BLK, _DIM), lambda i: (i, 0)),
            pl.BlockSpec((1, 1, _BLK), lambda i: (i, 0, 0)),
            pl.BlockSpec((1, 1), lambda i: (0, 0)),
        ],
        out_shape=[
            jax.ShapeDtypeStruct((_NTOK, _DIM), jnp.float32),
            jax.ShapeDtypeStruct((_NBLK, 1, _BLK), jnp.int32),
            jax.ShapeDtypeStruct((1, 1), jnp.float32),
        ],
    )(zf, wt, W, zsq, wsq[None, :])

    indices = idx3.reshape(_NTOK)
    loss = loss_sum[0, 0] * ((1.0 + _BETA) / float(_NTOK * _DIM))
    z_q = jnp.transpose(zq_flat.reshape(zp.shape), (0, 3, 1, 2))
    return (z_q, loss, indices)


# speedup vs baseline: 1.0661x; 1.0661x over previous
import jax
import jax.numpy as jnp
from jax import lax
from jax.experimental import pallas as pl

_CODEBOOK = 1024
_DIM = 256
_BETA = 0.25

_NB = 8
_HW = 1024
_NTOK = _NB * _HW


def _vq_body(zb_ref, w_ref, zsq_ref, wsq_ref, zq_ref, idx_ref, loss_ref):
    i = pl.program_id(0)
    zb = zb_ref[0]
    mm = lax.dot_general(w_ref[...], zb, (((1,), (0,)), ((), ())),
                         preferred_element_type=jnp.float32)
    d = (wsq_ref[...] + zsq_ref[0]) - 2.0 * mm
    dmin = jnp.min(d, axis=0)
    iota = lax.broadcasted_iota(jnp.int32, (_CODEBOOK, _HW), 0)
    idx = jnp.min(jnp.where(d == dmin[None, :], iota, _CODEBOOK), axis=0)
    idx_ref[...] = idx.reshape(1, 1, _HW)

    @pl.when(i == 0)
    def _():
        loss_ref[...] = jnp.zeros_like(loss_ref)

    loss_ref[...] += jnp.sum(dmin).reshape(1, 1)

    onehot = (iota == idx[None, :]).astype(jnp.float32)
    zq_ref[0] = lax.dot_general(w_ref[...], onehot, (((0,), (0,)), ((), ())),
                                preferred_element_type=jnp.float32,
                                precision=lax.Precision.HIGHEST)


@jax.jit
def kernel(z, W):
    zr = z.reshape(_NB, _DIM, _HW)
    zf = jnp.transpose(z, (0, 2, 3, 1)).reshape(-1, _DIM)
    zsq = jnp.sum(zf ** 2, axis=1).reshape(_NB, 1, _HW)
    wsq = jnp.sum(W ** 2, axis=1)

    zq, idx3, loss_sum = pl.pallas_call(
        _vq_body,
        grid=(_NB,),
        in_specs=[
            pl.BlockSpec((1, _DIM, _HW), lambda i: (i, 0, 0)),
            pl.BlockSpec((_CODEBOOK, _DIM), lambda i: (0, 0)),
            pl.BlockSpec((1, 1, _HW), lambda i: (i, 0, 0)),
            pl.BlockSpec((_CODEBOOK, 1), lambda i: (0, 0)),
        ],
        out_specs=[
            pl.BlockSpec((1, _DIM, _HW), lambda i: (i, 0, 0)),
            pl.BlockSpec((1, 1, _HW), lambda i: (i, 0, 0)),
            pl.BlockSpec((1, 1), lambda i: (0, 0)),
        ],
        out_shape=[
            jax.ShapeDtypeStruct((_NB, _DIM, _HW), jnp.float32),
            jax.ShapeDtypeStruct((_NB, 1, _HW), jnp.int32),
            jax.ShapeDtypeStruct((1, 1), jnp.float32),
        ],
    )(zr, W, zsq, wsq[:, None])

    indices = idx3.reshape(_NTOK)
    loss = loss_sum[0, 0] * ((1.0 + _BETA) / float(_NTOK * _DIM))
    z_q = zq.reshape(z.shape)
    return (z_q, loss, indices)


# speedup vs baseline: 1.5603x; 1.4636x over previous
import jax
import jax.numpy as jnp
from jax import lax
from jax.experimental import pallas as pl

_CODEBOOK = 1024
_DIM = 256
_BETA = 0.25

_NB = 8
_HW = 1024
_NTOK = _NB * _HW


def _vq_body(zb_ref, w_ref, zsq_ref, wsq_ref, zq_ref, idx_ref, loss_ref):
    i = pl.program_id(0)
    zb = zb_ref[0]
    mm = lax.dot_general(w_ref[...], zb, (((1,), (0,)), ((), ())),
                         preferred_element_type=jnp.float32)
    d = (wsq_ref[...] + zsq_ref[0]) - 2.0 * mm
    dmin = jnp.min(d, axis=0)
    iota = lax.broadcasted_iota(jnp.int32, (_CODEBOOK, _HW), 0)
    idx = jnp.min(jnp.where(d == dmin[None, :], iota, _CODEBOOK), axis=0)
    idx_ref[...] = idx.reshape(1, 1, _HW)

    @pl.when(i == 0)
    def _():
        loss_ref[...] = jnp.zeros_like(loss_ref)

    loss_ref[...] += jnp.sum(dmin).reshape(1, 1)

    onehot = (iota == idx[None, :]).astype(jnp.float32)
    zq_ref[0] = lax.dot_general(w_ref[...], onehot, (((0,), (0,)), ((), ())),
                                preferred_element_type=jnp.float32)


@jax.jit
def kernel(z, W):
    zr = z.reshape(_NB, _DIM, _HW)
    zf = jnp.transpose(z, (0, 2, 3, 1)).reshape(-1, _DIM)
    zsq = jnp.sum(zf ** 2, axis=1).reshape(_NB, 1, _HW)
    wsq = jnp.sum(W ** 2, axis=1)

    zq, idx3, loss_sum = pl.pallas_call(
        _vq_body,
        grid=(_NB,),
        in_specs=[
            pl.BlockSpec((1, _DIM, _HW), lambda i: (i, 0, 0)),
            pl.BlockSpec((_CODEBOOK, _DIM), lambda i: (0, 0)),
            pl.BlockSpec((1, 1, _HW), lambda i: (i, 0, 0)),
            pl.BlockSpec((_CODEBOOK, 1), lambda i: (0, 0)),
        ],
        out_specs=[
            pl.BlockSpec((1, _DIM, _HW), lambda i: (i, 0, 0)),
            pl.BlockSpec((1, 1, _HW), lambda i: (i, 0, 0)),
            pl.BlockSpec((1, 1), lambda i: (0, 0)),
        ],
        out_shape=[
            jax.ShapeDtypeStruct((_NB, _DIM, _HW), jnp.float32),
            jax.ShapeDtypeStruct((_NB, 1, _HW), jnp.int32),
            jax.ShapeDtypeStruct((1, 1), jnp.float32),
        ],
    )(zr, W, zsq, wsq[:, None])

    indices = idx3.reshape(_NTOK)
    loss = loss_sum[0, 0] * ((1.0 + _BETA) / float(_NTOK * _DIM))
    z_q = zq.reshape(z.shape)
    return (z_q, loss, indices)
